# baseline (speedup 1.0000x reference)
import functools

import jax
import jax.numpy as jnp
from jax import lax
from jax.experimental import pallas as pl
from jax.experimental.pallas import tpu as pltpu
from jax.experimental.pallas import tpu_sc as plsc

_NC = 2
_NS = 16
_NT = _NC * _NS
_L = 16


def _build_sc_scatter(n_nodes, n_in, n_edges, n_part, rng, sb, gb):
    npad = _NT * rng
    chunk = n_edges // n_part
    nsb = chunk // sb
    kin = n_in // _L

    mesh = plsc.VectorSubcoreMesh(
        core_axis_name="c", subcore_axis_name="s", num_cores=_NC,
        num_subcores=_NS)

    @functools.partial(
        pl.kernel,
        out_type=jax.ShapeDtypeStruct((n_part, npad * n_in), jnp.float32),
        mesh=mesh,
        compiler_params=pltpu.CompilerParams(needs_layout_passes=False),
        scratch_types=[
            pltpu.VMEM((sb,), jnp.int32),
            pltpu.VMEM((sb,), jnp.int32),
            pltpu.VMEM((sb,), jnp.float32),
            pltpu.VMEM((sb + _L,), jnp.int32),
            pltpu.VMEM((gb, n_in), jnp.float32),
            pltpu.VMEM((gb, n_in), jnp.float32),
            pltpu.VMEM((rng * n_in,), jnp.float32),
            pltpu.VMEM((gb,), jnp.int32),
            pltpu.VMEM((gb,), jnp.int32),
            pltpu.SemaphoreType.DMA,
            pltpu.SemaphoreType.DMA,
        ],
    )
    def sc_scatter(x_hbm, src_hbm, dst_hbm, w_hbm, agg_hbm,
                   dst_st, src_st, w_st, ids, rows0, rows1, acc,
                   gidx0, gidx1, sem0, sem1):
        cid = lax.axis_index("c")
        sid = lax.axis_index("s")
        wid = sid * _NC + cid
        lo = wid * rng
        lo_v = jnp.full((_L,), lo, jnp.int32)
        hi_v = lo_v + jnp.full((_L,), rng, jnp.int32)
        lane = lax.iota(jnp.int32, _L)
        zero_i = jnp.zeros((_L,), jnp.int32)
        zero_f = jnp.zeros((_L,), jnp.float32)

        def part_body(j, _):
            def zbody(i, _z):
                acc[pl.ds(i * _L, _L)] = jnp.zeros((_L,), jnp.float32)
                return 0
            lax.fori_loop(0, rng * kin, zbody, 0)

            def sb_body(b, _s):
                off = j * chunk + b * sb
                pltpu.sync_copy(dst_hbm.at[pl.ds(off, sb)], dst_st)
                pltpu.sync_copy(src_hbm.at[pl.ds(off, sb)], src_st)
                pltpu.sync_copy(w_hbm.at[pl.ds(off, sb)], w_st)

                def fbody(i, cnt_v):
                    dv = dst_st[pl.ds(i * _L, _L)]
                    m = (dv >= lo_v) & (dv < hi_v)
                    pos = cnt_v + plsc.cumsum(m.astype(jnp.int32)) - \
                        jnp.full((_L,), 1, jnp.int32)
                    idv = lane + jnp.full((_L,), i * _L, jnp.int32)
                    plsc.store_scatter(ids, [pos], idv, mask=m)
                    return cnt_v + plsc.all_reduce_population_count(m)

                cnt_v = lax.fori_loop(0, sb // _L, fbody,
                                      jnp.zeros((_L,), jnp.int32))
                cnt = cnt_v[0]

                def build(b32, slot):
                    for h in range(gb // _L):
                        gl = lane + jnp.full((_L,), h * _L, jnp.int32) + \
                            b32 * jnp.full((_L,), gb, jnp.int32)
                        valid = gl < cnt_v
                        idv = plsc.load_gather(
                            ids, [jnp.where(valid, gl, zero_i)])
                        idv = jnp.where(valid, idv, zero_i)
                        srcv = plsc.load_gather(src_st, [idv])
                        slot[pl.ds(h * _L, _L)] = srcv

                def accum(b32, rows):
                    for h in range(gb // _L):
                        gl = lane + jnp.full((_L,), h * _L, jnp.int32) + \
                            b32 * jnp.full((_L,), gb, jnp.int32)
                        valid = gl < cnt_v
                        idv = plsc.load_gather(
                            ids, [jnp.where(valid, gl, zero_i)])
                        idv = jnp.where(valid, idv, zero_i)
                        wv = jnp.where(valid,
                                       plsc.load_gather(w_st, [idv]), zero_f)
                        dvv = plsc.load_gather(dst_st, [idv])
                        rowv = jnp.where(valid, dvv - lo_v, zero_i)
                        for e in range(_L):
                            ws = jnp.full((_L,), wv[e], jnp.float32)
                            base = rowv[e] * n_in
                            for k in range(kin):
                                gvec = rows[h * _L + e, pl.ds(k * _L, _L)]
                                plsc.addupdate(
                                    acc.at[pl.ds(base + k * _L, _L)],
                                    gvec * ws)

                nb = (cnt + gb - 1) // gb
                np2 = (nb + 1) // 2

                build(0, gidx0)
                pltpu.async_copy(x_hbm.at[gidx0], rows0, sem0)

                def pair_body(p, _p):
                    pltpu.make_async_copy(x_hbm.at[gidx0], rows0,
                                          sem0).wait()
                    build(2 * p + 1, gidx1)
                    pltpu.async_copy(x_hbm.at[gidx1], rows1, sem1)
                    accum(2 * p, rows0)
                    pltpu.make_async_copy(x_hbm.at[gidx1], rows1,
                                          sem1).wait()
                    build(2 * p + 2, gidx0)
                    pltpu.async_copy(x_hbm.at[gidx0], rows0, sem0)
                    accum(2 * p + 1, rows1)
                    return 0

                lax.fori_loop(0, np2, pair_body, 0)
                pltpu.make_async_copy(x_hbm.at[gidx0], rows0, sem0).wait()
                return 0

            lax.fori_loop(0, nsb, sb_body, 0)
            pltpu.sync_copy(acc, agg_hbm.at[j, pl.ds(lo * n_in, rng * n_in)])
            return 0

        lax.fori_loop(0, n_part, part_body, 0)

    return sc_scatter


def _build_tc_matmul(npad, n_in, n_out, n_part, bm):
    def mm_body(a_ref, w_ref, b_ref, o_ref):
        acc = jnp.zeros((bm, n_out), jnp.float32)
        for j in range(n_part):
            acc = acc + jnp.dot(a_ref[j], w_ref[j],
                                preferred_element_type=jnp.float32)
        o_ref[...] = jnp.maximum(acc + b_ref[...], 0.0)

    return pl.pallas_call(
        mm_body,
        grid=(npad // bm,),
        in_specs=[
            pl.BlockSpec((n_part, bm, n_in), lambda i: (0, i, 0)),
            pl.BlockSpec((n_part, n_in, n_out), lambda i: (0, 0, 0)),
            pl.BlockSpec((1, n_out), lambda i: (0, 0)),
        ],
        out_specs=pl.BlockSpec((bm, n_out), lambda i: (i, 0)),
        out_shape=jax.ShapeDtypeStruct((npad, n_out), jnp.float32),
    )


def kernel(x, edge_index, edge_weight, weights, bias):
    n_nodes, n_in = x.shape
    n_part, _, n_out = weights.shape
    n_edges = edge_weight.shape[0]

    rng = -(-n_nodes // _NT)
    rng = -(-rng // 8) * 8
    npad = _NT * rng
    sb = 4000

    src = edge_index[0].astype(jnp.int32)
    dst = edge_index[1].astype(jnp.int32)
    w = edge_weight.astype(jnp.float32)

    sc = _build_sc_scatter(n_nodes, n_in, n_edges, n_part, rng, sb, gb=32)
    agg = sc(x, src, dst, w)
    agg3 = agg.reshape(n_part, npad, n_in)

    mm = _build_tc_matmul(npad, n_in, n_out, n_part, bm=1024)
    out = mm(agg3, weights, bias.reshape(1, n_out))
    return out[:n_nodes]


# speedup vs baseline: 1.1712x; 1.1712x over previous
import functools

import jax
import jax.numpy as jnp
from jax import lax
from jax.experimental import pallas as pl
from jax.experimental.pallas import tpu as pltpu
from jax.experimental.pallas import tpu_sc as plsc

_NC = 2
_NS = 16
_NT = _NC * _NS
_L = 16


def _build_sc_scatter(n_nodes, n_in, n_edges, n_part, rng, sb, gb):
    npad = _NT * rng
    chunk = n_edges // n_part
    nsb = chunk // sb
    kin = n_in // _L

    mesh = plsc.VectorSubcoreMesh(
        core_axis_name="c", subcore_axis_name="s", num_cores=_NC,
        num_subcores=_NS)

    @functools.partial(
        pl.kernel,
        out_type=jax.ShapeDtypeStruct((n_part, npad * n_in), jnp.float32),
        mesh=mesh,
        compiler_params=pltpu.CompilerParams(needs_layout_passes=False),
        scratch_types=[
            pltpu.VMEM((sb,), jnp.int32),
            pltpu.VMEM((sb,), jnp.int32),
            pltpu.VMEM((sb,), jnp.float32),
            pltpu.VMEM((sb + _L,), jnp.int32),
            pltpu.VMEM((gb, n_in), jnp.float32),
            pltpu.VMEM((gb, n_in), jnp.float32),
            pltpu.VMEM((rng * n_in,), jnp.float32),
            pltpu.VMEM((gb,), jnp.int32),
            pltpu.VMEM((gb,), jnp.int32),
            pltpu.SemaphoreType.DMA,
            pltpu.SemaphoreType.DMA,
        ],
    )
    def sc_scatter(x_hbm, src_hbm, dst_hbm, w_hbm, agg_hbm,
                   dst_st, src_st, w_st, ids, rows0, rows1, acc,
                   gidx0, gidx1, sem0, sem1):
        cid = lax.axis_index("c")
        sid = lax.axis_index("s")
        wid = sid * _NC + cid
        lo = wid * rng
        lo_v = jnp.full((_L,), lo, jnp.int32)
        hi_v = lo_v + jnp.full((_L,), rng, jnp.int32)
        lane = lax.iota(jnp.int32, _L)
        zero_i = jnp.zeros((_L,), jnp.int32)
        zero_f = jnp.zeros((_L,), jnp.float32)

        def part_body(j, _):
            def zbody(i, _z):
                acc[pl.ds(i * _L, _L)] = jnp.zeros((_L,), jnp.float32)
                return 0
            lax.fori_loop(0, rng * kin, zbody, 0)

            def sb_body(b, _s):
                off = j * chunk + b * sb
                pltpu.sync_copy(dst_hbm.at[pl.ds(off, sb)], dst_st)
                pltpu.sync_copy(src_hbm.at[pl.ds(off, sb)], src_st)
                pltpu.sync_copy(w_hbm.at[pl.ds(off, sb)], w_st)

                def fbody(i, cnt_v):
                    dv = dst_st[pl.ds(i * _L, _L)]
                    m = (dv >= lo_v) & (dv < hi_v)
                    pos = cnt_v + plsc.cumsum(m.astype(jnp.int32)) - \
                        jnp.full((_L,), 1, jnp.int32)
                    idv = lane + jnp.full((_L,), i * _L, jnp.int32)
                    plsc.store_scatter(ids, [pos], idv, mask=m)
                    return cnt_v + plsc.all_reduce_population_count(m)

                cnt_v = lax.fori_loop(0, sb // _L, fbody,
                                      jnp.zeros((_L,), jnp.int32))
                cnt = cnt_v[0]

                def build(b32, slot):
                    for h in range(gb // _L):
                        gl = lane + jnp.full((_L,), h * _L, jnp.int32) + \
                            b32 * jnp.full((_L,), gb, jnp.int32)
                        valid = gl < cnt_v
                        idv = plsc.load_gather(
                            ids, [jnp.where(valid, gl, zero_i)])
                        idv = jnp.where(valid, idv, zero_i)
                        srcv = plsc.load_gather(src_st, [idv])
                        slot[pl.ds(h * _L, _L)] = srcv

                def accum(b32, rows):
                    for h in range(gb // _L):
                        gl = lane + jnp.full((_L,), h * _L, jnp.int32) + \
                            b32 * jnp.full((_L,), gb, jnp.int32)
                        valid = gl < cnt_v
                        idv = plsc.load_gather(
                            ids, [jnp.where(valid, gl, zero_i)])
                        idv = jnp.where(valid, idv, zero_i)
                        wv = jnp.where(valid,
                                       plsc.load_gather(w_st, [idv]), zero_f)
                        dvv = plsc.load_gather(dst_st, [idv])
                        rowv = jnp.where(valid, dvv - lo_v, zero_i)
                        for e in range(_L):
                            ws = jnp.full((_L,), wv[e], jnp.float32)
                            base = rowv[e] * n_in
                            for k in range(kin):
                                gvec = rows[h * _L + e, pl.ds(k * _L, _L)]
                                plsc.addupdate(
                                    acc.at[pl.ds(base + k * _L, _L)],
                                    gvec * ws)

                def gbody(g, _g):
                    build(g, gidx0)
                    pltpu.async_copy(x_hbm.at[gidx0], rows0, sem0).wait()
                    accum(g, rows0)
                    return 0

                lax.fori_loop(0, (cnt + gb - 1) // gb, gbody, 0)
                return 0

            lax.fori_loop(0, nsb, sb_body, 0)
            pltpu.sync_copy(acc, agg_hbm.at[j, pl.ds(lo * n_in, rng * n_in)])
            return 0

        lax.fori_loop(0, n_part, part_body, 0)

    return sc_scatter


def _build_tc_matmul(npad, n_in, n_out, n_part, bm):
    def mm_body(a_ref, w_ref, b_ref, o_ref):
        acc = jnp.zeros((bm, n_out), jnp.float32)
        for j in range(n_part):
            acc = acc + jnp.dot(a_ref[j], w_ref[j],
                                preferred_element_type=jnp.float32)
        o_ref[...] = jnp.maximum(acc + b_ref[...], 0.0)

    return pl.pallas_call(
        mm_body,
        grid=(npad // bm,),
        in_specs=[
            pl.BlockSpec((n_part, bm, n_in), lambda i: (0, i, 0)),
            pl.BlockSpec((n_part, n_in, n_out), lambda i: (0, 0, 0)),
            pl.BlockSpec((1, n_out), lambda i: (0, 0)),
        ],
        out_specs=pl.BlockSpec((bm, n_out), lambda i: (i, 0)),
        out_shape=jax.ShapeDtypeStruct((npad, n_out), jnp.float32),
    )


def kernel(x, edge_index, edge_weight, weights, bias):
    n_nodes, n_in = x.shape
    n_part, _, n_out = weights.shape
    n_edges = edge_weight.shape[0]

    rng = -(-n_nodes // _NT)
    rng = -(-rng // 8) * 8
    npad = _NT * rng
    sb = 4000

    src = edge_index[0].astype(jnp.int32)
    dst = edge_index[1].astype(jnp.int32)
    w = edge_weight.astype(jnp.float32)

    sc = _build_sc_scatter(n_nodes, n_in, n_edges, n_part, rng, sb, gb=32)
    agg = sc(x, src, dst, w)
    agg3 = agg.reshape(n_part, npad, n_in)

    mm = _build_tc_matmul(npad, n_in, n_out, n_part, bm=1024)
    out = mm(agg3, weights, bias.reshape(1, n_out))
    return out[:n_nodes]


# speedup vs baseline: 1.3344x; 1.1393x over previous
import functools

import jax
import jax.numpy as jnp
from jax import lax
from jax.experimental import pallas as pl
from jax.experimental.pallas import tpu as pltpu
from jax.experimental.pallas import tpu_sc as plsc

_NC = 2
_NS = 16
_NT = _NC * _NS
_L = 16


def _build_sc_scatter(n_nodes, n_in, n_edges, n_part, rng, sb, gb):
    npad = _NT * rng
    chunk = n_edges // n_part
    nsb = chunk // sb
    kin = n_in // _L

    mesh = plsc.VectorSubcoreMesh(
        core_axis_name="c", subcore_axis_name="s", num_cores=_NC,
        num_subcores=_NS)

    @functools.partial(
        pl.kernel,
        out_type=jax.ShapeDtypeStruct((n_part, npad * n_in), jnp.float32),
        mesh=mesh,
        compiler_params=pltpu.CompilerParams(needs_layout_passes=False),
        scratch_types=[
            pltpu.VMEM((sb,), jnp.int32),
            pltpu.VMEM((sb,), jnp.int32),
            pltpu.VMEM((sb,), jnp.float32),
            pltpu.VMEM((sb + _L,), jnp.int32),
            pltpu.VMEM((gb, n_in), jnp.float32),
            pltpu.VMEM((gb, n_in), jnp.float32),
            pltpu.VMEM((rng * n_in,), jnp.float32),
            pltpu.VMEM((gb,), jnp.int32),
            pltpu.VMEM((gb,), jnp.int32),
            pltpu.SemaphoreType.DMA,
            pltpu.SemaphoreType.DMA,
        ],
    )
    def sc_scatter(x_hbm, src_hbm, dst_hbm, w_hbm, agg_hbm,
                   dst_st, src_st, w_st, ids, rows0, rows1, acc,
                   gidx0, gidx1, sem0, sem1):
        cid = lax.axis_index("c")
        sid = lax.axis_index("s")
        wid = sid * _NC + cid
        lo = wid * rng
        lo_v = jnp.full((_L,), lo, jnp.int32)
        hi_v = lo_v + jnp.full((_L,), rng, jnp.int32)
        lane = lax.iota(jnp.int32, _L)
        zero_i = jnp.zeros((_L,), jnp.int32)
        zero_f = jnp.zeros((_L,), jnp.float32)

        def part_body(j, _):
            def zbody(i, _z):
                acc[pl.ds(i * _L, _L)] = jnp.zeros((_L,), jnp.float32)
                return 0
            lax.fori_loop(0, rng * kin, zbody, 0)

            def sb_body(b, _s):
                off = j * chunk + b * sb
                pltpu.sync_copy(dst_hbm.at[pl.ds(off, sb)], dst_st)
                pltpu.sync_copy(src_hbm.at[pl.ds(off, sb)], src_st)
                pltpu.sync_copy(w_hbm.at[pl.ds(off, sb)], w_st)

                one_v = jnp.full((_L,), 1, jnp.int32)

                def fbody(i, cnt_v):
                    dv0 = dst_st[pl.ds((2 * i) * _L, _L)]
                    dv1 = dst_st[pl.ds((2 * i + 1) * _L, _L)]
                    m0 = (dv0 >= lo_v) & (dv0 < hi_v)
                    m1 = (dv1 >= lo_v) & (dv1 < hi_v)
                    cs0 = plsc.cumsum(m0.astype(jnp.int32))
                    cs1 = plsc.cumsum(m1.astype(jnp.int32))
                    pc0 = plsc.all_reduce_population_count(m0)
                    pc1 = plsc.all_reduce_population_count(m1)
                    id0 = lane + jnp.full((_L,), 2 * i * _L, jnp.int32)
                    id1 = id0 + jnp.full((_L,), _L, jnp.int32)
                    plsc.store_scatter(ids, [cnt_v + cs0 - one_v], id0,
                                       mask=m0)
                    plsc.store_scatter(ids, [cnt_v + pc0 + cs1 - one_v],
                                       id1, mask=m1)
                    return cnt_v + pc0 + pc1

                cnt_v = lax.fori_loop(0, sb // (2 * _L), fbody,
                                      jnp.zeros((_L,), jnp.int32))
                cnt = cnt_v[0]

                def gbody(g, _g):
                    gl = lane + jnp.full((_L,), g * _L, jnp.int32)
                    valid = gl < cnt_v
                    idv = plsc.load_gather(ids, [jnp.where(valid, gl,
                                                           zero_i)])
                    idv = jnp.where(valid, idv, zero_i)
                    srcv = plsc.load_gather(src_st, [idv])
                    wv = jnp.where(valid, plsc.load_gather(w_st, [idv]),
                                   zero_f)
                    dvv = plsc.load_gather(dst_st, [idv])
                    rowv = jnp.where(valid, dvv - lo_v, zero_i)
                    gidx0[pl.ds(0, _L)] = srcv
                    pltpu.async_copy(x_hbm.at[gidx0], rows0, sem0).wait()
                    ws_l = [jnp.full((_L,), wv[e], jnp.float32)
                            for e in range(_L)]
                    base_l = [rowv[e] * n_in for e in range(_L)]
                    for k in range(kin):
                        for e in range(_L):
                            gvec = rows0[e, pl.ds(k * _L, _L)]
                            plsc.addupdate(
                                acc.at[pl.ds(base_l[e] + k * _L, _L)],
                                gvec * ws_l[e])
                    return 0

                lax.fori_loop(0, (cnt + _L - 1) // _L, gbody, 0)
                return 0

            lax.fori_loop(0, nsb, sb_body, 0)
            pltpu.sync_copy(acc, agg_hbm.at[j, pl.ds(lo * n_in, rng * n_in)])
            return 0

        lax.fori_loop(0, n_part, part_body, 0)

    return sc_scatter


def _build_tc_matmul(npad, n_in, n_out, n_part, bm):
    def mm_body(a_ref, w_ref, b_ref, o_ref):
        acc = jnp.zeros((bm, n_out), jnp.float32)
        for j in range(n_part):
            acc = acc + jnp.dot(a_ref[j], w_ref[j],
                                preferred_element_type=jnp.float32)
        o_ref[...] = jnp.maximum(acc + b_ref[...], 0.0)

    return pl.pallas_call(
        mm_body,
        grid=(npad // bm,),
        in_specs=[
            pl.BlockSpec((n_part, bm, n_in), lambda i: (0, i, 0)),
            pl.BlockSpec((n_part, n_in, n_out), lambda i: (0, 0, 0)),
            pl.BlockSpec((1, n_out), lambda i: (0, 0)),
        ],
        out_specs=pl.BlockSpec((bm, n_out), lambda i: (i, 0)),
        out_shape=jax.ShapeDtypeStruct((npad, n_out), jnp.float32),
    )


def kernel(x, edge_index, edge_weight, weights, bias):
    n_nodes, n_in = x.shape
    n_part, _, n_out = weights.shape
    n_edges = edge_weight.shape[0]

    rng = -(-n_nodes // _NT)
    rng = -(-rng // 8) * 8
    npad = _NT * rng
    sb = 4000

    src = edge_index[0].astype(jnp.int32)
    dst = edge_index[1].astype(jnp.int32)
    w = edge_weight.astype(jnp.float32)

    sc = _build_sc_scatter(n_nodes, n_in, n_edges, n_part, rng, sb, gb=16)
    agg = sc(x, src, dst, w)
    agg3 = agg.reshape(n_part, npad, n_in)

    mm = _build_tc_matmul(npad, n_in, n_out, n_part, bm=1024)
    out = mm(agg3, weights, bias.reshape(1, n_out))
    return out[:n_nodes]


# speedup vs baseline: 1.3346x; 1.0002x over previous
import functools

import jax
import jax.numpy as jnp
from jax import lax
from jax.experimental import pallas as pl
from jax.experimental.pallas import tpu as pltpu
from jax.experimental.pallas import tpu_sc as plsc

_NC = 2
_NS = 16
_NT = _NC * _NS
_L = 16


def _build_sc_scatter(n_nodes, n_in, n_edges, n_part, rng, sb, gb):
    npad = _NT * rng
    chunk = n_edges // n_part
    nsb = chunk // sb
    kin = n_in // _L

    mesh = plsc.VectorSubcoreMesh(
        core_axis_name="c", subcore_axis_name="s", num_cores=_NC,
        num_subcores=_NS)

    @functools.partial(
        pl.kernel,
        out_type=jax.ShapeDtypeStruct((n_part, npad * n_in), jnp.float32),
        mesh=mesh,
        compiler_params=pltpu.CompilerParams(needs_layout_passes=False),
        scratch_types=[
            pltpu.VMEM((sb,), jnp.int32),
            pltpu.VMEM((sb,), jnp.int32),
            pltpu.VMEM((sb,), jnp.float32),
            pltpu.VMEM((sb + _L,), jnp.int32),
            pltpu.VMEM((gb, n_in), jnp.float32),
            pltpu.VMEM((gb, n_in), jnp.float32),
            pltpu.VMEM((rng * n_in,), jnp.float32),
            pltpu.VMEM((gb,), jnp.int32),
            pltpu.VMEM((gb,), jnp.int32),
            pltpu.SemaphoreType.DMA,
            pltpu.SemaphoreType.DMA,
        ],
    )
    def sc_scatter(x_hbm, src_hbm, dst_hbm, w_hbm, agg_hbm,
                   dst_st, src_st, w_st, ids, rows0, rows1, acc,
                   gidx0, gidx1, sem0, sem1):
        cid = lax.axis_index("c")
        sid = lax.axis_index("s")
        wid = sid * _NC + cid
        lo = wid * rng
        lo_v = jnp.full((_L,), lo, jnp.int32)
        hi_v = lo_v + jnp.full((_L,), rng, jnp.int32)
        lane = lax.iota(jnp.int32, _L)
        zero_i = jnp.zeros((_L,), jnp.int32)
        zero_f = jnp.zeros((_L,), jnp.float32)

        def part_body(j, _):
            def zbody(i, _z):
                acc[pl.ds(i * _L, _L)] = jnp.zeros((_L,), jnp.float32)
                return 0
            lax.fori_loop(0, rng * kin, zbody, 0)

            def sb_body(b, _s):
                off = j * chunk + b * sb
                pltpu.sync_copy(dst_hbm.at[pl.ds(off, sb)], dst_st)
                pltpu.sync_copy(src_hbm.at[pl.ds(off, sb)], src_st)
                pltpu.sync_copy(w_hbm.at[pl.ds(off, sb)], w_st)

                one_v = jnp.full((_L,), 1, jnp.int32)

                def fbody(i, cnt_v):
                    dv0 = dst_st[pl.ds((2 * i) * _L, _L)]
                    dv1 = dst_st[pl.ds((2 * i + 1) * _L, _L)]
                    m0 = (dv0 >= lo_v) & (dv0 < hi_v)
                    m1 = (dv1 >= lo_v) & (dv1 < hi_v)
                    cs0 = plsc.cumsum(m0.astype(jnp.int32))
                    cs1 = plsc.cumsum(m1.astype(jnp.int32))
                    pc0 = plsc.all_reduce_population_count(m0)
                    pc1 = plsc.all_reduce_population_count(m1)
                    id0 = lane + jnp.full((_L,), 2 * i * _L, jnp.int32)
                    id1 = id0 + jnp.full((_L,), _L, jnp.int32)
                    plsc.store_scatter(ids, [cnt_v + cs0 - one_v], id0,
                                       mask=m0)
                    plsc.store_scatter(ids, [cnt_v + pc0 + cs1 - one_v],
                                       id1, mask=m1)
                    return cnt_v + pc0 + pc1

                cnt_v = lax.fori_loop(0, sb // (2 * _L), fbody,
                                      jnp.zeros((_L,), jnp.int32))
                cnt = cnt_v[0]

                def gbody(g, _g):
                    gl = lane + jnp.full((_L,), g * _L, jnp.int32)
                    valid = gl < cnt_v
                    idv = plsc.load_gather(ids, [jnp.where(valid, gl,
                                                           zero_i)])
                    idv = jnp.where(valid, idv, zero_i)
                    srcv = plsc.load_gather(src_st, [idv])
                    gidx0[pl.ds(0, _L)] = srcv
                    cp = pltpu.async_copy(x_hbm.at[gidx0], rows0, sem0)
                    wv = jnp.where(valid, plsc.load_gather(w_st, [idv]),
                                   zero_f)
                    dvv = plsc.load_gather(dst_st, [idv])
                    rowv = jnp.where(valid, dvv - lo_v, zero_i)
                    ws_l = [jnp.full((_L,), wv[e], jnp.float32)
                            for e in range(_L)]
                    base_l = [rowv[e] * n_in for e in range(_L)]
                    cp.wait()
                    for k in range(kin):
                        for e in range(_L):
                            gvec = rows0[e, pl.ds(k * _L, _L)]
                            plsc.addupdate(
                                acc.at[pl.ds(base_l[e] + k * _L, _L)],
                                gvec * ws_l[e])
                    return 0

                lax.fori_loop(0, (cnt + _L - 1) // _L, gbody, 0)
                return 0

            lax.fori_loop(0, nsb, sb_body, 0)
            pltpu.sync_copy(acc, agg_hbm.at[j, pl.ds(lo * n_in, rng * n_in)])
            return 0

        lax.fori_loop(0, n_part, part_body, 0)

    return sc_scatter


def _build_tc_matmul(npad, n_in, n_out, n_part, bm):
    def mm_body(a_ref, w_ref, b_ref, o_ref):
        acc = jnp.zeros((bm, n_out), jnp.float32)
        for j in range(n_part):
            acc = acc + jnp.dot(a_ref[j], w_ref[j],
                                preferred_element_type=jnp.float32)
        o_ref[...] = jnp.maximum(acc + b_ref[...], 0.0)

    return pl.pallas_call(
        mm_body,
        grid=(npad // bm,),
        in_specs=[
            pl.BlockSpec((n_part, bm, n_in), lambda i: (0, i, 0)),
            pl.BlockSpec((n_part, n_in, n_out), lambda i: (0, 0, 0)),
            pl.BlockSpec((1, n_out), lambda i: (0, 0)),
        ],
        out_specs=pl.BlockSpec((bm, n_out), lambda i: (i, 0)),
        out_shape=jax.ShapeDtypeStruct((npad, n_out), jnp.float32),
    )


def kernel(x, edge_index, edge_weight, weights, bias):
    n_nodes, n_in = x.shape
    n_part, _, n_out = weights.shape
    n_edges = edge_weight.shape[0]

    rng = -(-n_nodes // _NT)
    rng = -(-rng // 8) * 8
    npad = _NT * rng
    sb = 4000

    src = edge_index[0].astype(jnp.int32)
    dst = edge_index[1].astype(jnp.int32)
    w = edge_weight.astype(jnp.float32)

    sc = _build_sc_scatter(n_nodes, n_in, n_edges, n_part, rng, sb, gb=16)
    agg = sc(x, src, dst, w)
    agg3 = agg.reshape(n_part, npad, n_in)

    mm = _build_tc_matmul(npad, n_in, n_out, n_part, bm=1024)
    out = mm(agg3, weights, bias.reshape(1, n_out))
    return out[:n_nodes]
